# baseline (speedup 1.0000x reference)
import jax
import jax.numpy as jnp
from jax.experimental import pallas as pl
from jax.experimental.pallas import tpu as pltpu

N_NODES = 10000
NUM_ATOM_TYPE = 119
MASK_RATIO = 0.15
NOISE_VAL = 0.1
EPS = 1e-5

_BE = 2000
_NM = 1500
_NMP = 1504


def _edge_chain_body(atoms_ref, ea_ref, pe2_ref, pn2_ref, hp_ref, a_ref,
                     b_ref, wdT_ref, bd_ref, g_ref, lb_ref, wout_ref,
                     bout_ref, out_ref):
    atoms = atoms_ref[...]
    cols = jax.lax.broadcasted_iota(jnp.int32, (_BE, 128), 1)
    onehot = jnp.where(cols == atoms, 1.0, 0.0)
    hp_src = jnp.dot(onehot, hp_ref[...], preferred_element_type=jnp.float32)
    pn = jnp.sqrt(pn2_ref[...])
    q = jnp.dot(ea_ref[...], a_ref[0:4, :],
                preferred_element_type=jnp.float32) + pn * b_ref[...]
    pe = jnp.maximum(hp_src + q, 0.0)
    t = jnp.dot(pe, wdT_ref[...],
                preferred_element_type=jnp.float32) + bd_ref[...]
    t = jax.nn.gelu(t)
    mu = jnp.mean(t, axis=-1, keepdims=True)
    var = jnp.mean((t - mu) * (t - mu), axis=-1, keepdims=True)
    tn = (t - mu) * jax.lax.rsqrt(var + EPS) * g_ref[...] + lb_ref[...]
    dout = jnp.sum(tn * wout_ref[...], axis=-1, keepdims=True) + bout_ref[0, 0]
    out_ref[...] = dout - jnp.sqrt(pe2_ref[...])


def _edge_chain(atoms, edge_attr, pe2, pn2, hp_pad, A, b_row, dh_dense_w,
                dh_dense_b, dh_ln_g, dh_ln_b, dh_out_w, dh_out_b):
    E = atoms.shape[0]
    D = 128
    grid = (E // _BE,)
    full = lambda i: (0, 0)
    blk = lambda i: (i, 0)
    return pl.pallas_call(
        _edge_chain_body,
        grid=grid,
        in_specs=[
            pl.BlockSpec((_BE, 1), blk),
            pl.BlockSpec((_BE, 4), blk),
            pl.BlockSpec((_BE, 1), blk),
            pl.BlockSpec((_BE, 1), blk),
            pl.BlockSpec((D, D), full),
            pl.BlockSpec((8, D), full),
            pl.BlockSpec((1, D), full),
            pl.BlockSpec((D, D), full),
            pl.BlockSpec((1, D), full),
            pl.BlockSpec((1, D), full),
            pl.BlockSpec((1, D), full),
            pl.BlockSpec((1, D), full),
            pl.BlockSpec((1, 1), full),
        ],
        out_specs=pl.BlockSpec((_BE, 1), blk),
        out_shape=jax.ShapeDtypeStruct((E, 1), jnp.float32),
    )(atoms.reshape(E, 1), edge_attr, pe2.reshape(E, 1), pn2.reshape(E, 1),
      hp_pad, A, b_row, dh_dense_w.T, dh_dense_b.reshape(1, D),
      dh_ln_g.reshape(1, D), dh_ln_b.reshape(1, D), dh_out_w.reshape(1, D),
      dh_out_b.reshape(1, 1))


def _node_head_body(cg_ref, eam_ref, pnm_ref, snm_ref, wemb_ref, wedge_ref,
                    wpe_ref, wgnn_ref, mdwT_ref, mdb_ref, mg_ref, mb_ref,
                    mwT_ref, mbias_ref, tgt_ref, out_ref):
    C = cg_ref[0] + cg_ref[1]
    Ea = eam_ref[0] + eam_ref[1]
    pn = pnm_ref[0] + pnm_ref[1]
    G = jnp.dot(C, wemb_ref[...], preferred_element_type=jnp.float32)
    EaW = jnp.dot(Ea[:, 0:4], wedge_ref[0:4, :],
                  preferred_element_type=jnp.float32)
    aggm = G + EaW + pn * wpe_ref[...]
    h0m = wemb_ref[NUM_ATOM_TYPE:NUM_ATOM_TYPE + 1, :]
    z = jnp.dot(h0m + aggm * snm_ref[...], wgnn_ref[...],
                preferred_element_type=jnp.float32)
    feats = jnp.maximum(z, 0.0)
    h = jnp.dot(feats, mdwT_ref[...],
                preferred_element_type=jnp.float32) + mdb_ref[...]
    h = jax.nn.gelu(h)
    mu = jnp.mean(h, axis=-1, keepdims=True)
    var = jnp.mean((h - mu) * (h - mu), axis=-1, keepdims=True)
    h = (h - mu) * jax.lax.rsqrt(var + EPS) * mg_ref[...] + mb_ref[...]
    pred = jnp.dot(h, mwT_ref[...],
                   preferred_element_type=jnp.float32) + mbias_ref[...]
    cols = jax.lax.broadcasted_iota(jnp.int32, (_NMP, 128), 1)
    rows = jax.lax.broadcasted_iota(jnp.int32, (_NMP, 128), 0)
    pred = jnp.where(cols < NUM_ATOM_TYPE, pred, -1e30)
    mx = jnp.max(pred, axis=-1, keepdims=True)
    lse = mx + jnp.log(jnp.sum(jnp.exp(pred - mx), axis=-1, keepdims=True))
    logp = pred - lse
    sel = (cols == tgt_ref[...]) & (rows < _NM)
    out_ref[...] = (-jnp.sum(jnp.where(sel, logp, 0.0)) / _NM).reshape(1, 1)


def _node_head(cg, eam, pnm, snm, wemb_pad, wedge_pad, w_pe, W_gnn,
               mlm_dense_w, mlm_dense_b, mlm_ln_g, mlm_ln_b, mlm_weightT_pad,
               mlm_bias_pad, tgt):
    D = 128
    full2 = lambda: None
    specs = [
        pl.BlockSpec((2, _NMP, D), lambda: (0, 0, 0)),
        pl.BlockSpec((2, _NMP, 8), lambda: (0, 0, 0)),
        pl.BlockSpec((2, _NMP, 1), lambda: (0, 0, 0)),
        pl.BlockSpec((_NMP, 1), lambda: (0, 0)),
        pl.BlockSpec((D, D), lambda: (0, 0)),
        pl.BlockSpec((8, D), lambda: (0, 0)),
        pl.BlockSpec((1, D), lambda: (0, 0)),
        pl.BlockSpec((D, D), lambda: (0, 0)),
        pl.BlockSpec((D, D), lambda: (0, 0)),
        pl.BlockSpec((1, D), lambda: (0, 0)),
        pl.BlockSpec((1, D), lambda: (0, 0)),
        pl.BlockSpec((1, D), lambda: (0, 0)),
        pl.BlockSpec((D, D), lambda: (0, 0)),
        pl.BlockSpec((1, D), lambda: (0, 0)),
        pl.BlockSpec((_NMP, 1), lambda: (0, 0)),
    ]
    return pl.pallas_call(
        _node_head_body,
        in_specs=specs,
        out_specs=pl.BlockSpec((1, 1), lambda: (0, 0)),
        out_shape=jax.ShapeDtypeStruct((1, 1), jnp.float32),
    )(cg, eam, pnm, snm, wemb_pad, wedge_pad, w_pe, W_gnn,
      mlm_dense_w.T, mlm_dense_b.reshape(1, D), mlm_ln_g.reshape(1, D),
      mlm_ln_b.reshape(1, D), mlm_weightT_pad, mlm_bias_pad,
      tgt.reshape(_NMP, 1))


def _huber(x):
    ax = jnp.abs(x)
    return jnp.where(ax < 1.0, 0.5 * x * x, ax - 0.5)


def kernel(x, edge_index, edge_attr, snorm_n, EigVals, EigVecs, W_embed,
           W_edge, w_pe, W_gnn, W_pe2, mlm_dense_w, mlm_dense_b, mlm_ln_g,
           mlm_ln_b, mlm_weight, mlm_bias, dh_dense_w, dh_dense_b, dh_ln_g,
           dh_ln_b, dh_out_w, dh_out_b):
    N = x.shape[0]
    E = edge_index.shape[1]
    u = jnp.nan_to_num(EigVecs)
    src = edge_index[0]
    dst = edge_index[1]

    mkey = jax.random.key(42)
    perm = jax.random.permutation(mkey, N)
    num_mask = int(MASK_RATIO * N)
    mask_nodes = perm[:num_mask]
    noise = NOISE_VAL * jax.random.normal(
        jax.random.fold_in(mkey, 1), (num_mask, u.shape[1]),
        dtype=jnp.float32)
    node_is_masked = jnp.zeros((N,), bool).at[mask_nodes].set(True)
    lut = jnp.full((N,), -1, jnp.int32).at[mask_nodes].set(
        jnp.arange(num_mask, dtype=jnp.int32))

    u_masked = u.at[mask_nodes].add(noise)
    xm0 = jnp.where(node_is_masked, NUM_ATOM_TYPE, x[:, 0]).astype(jnp.int32)

    wemb_pad = jnp.zeros((128, 128), jnp.float32).at[:NUM_ATOM_TYPE + 1].set(
        W_embed)
    hp_pad = wemb_pad @ W_pe2
    A = jnp.zeros((8, 128), jnp.float32).at[:4].set(W_edge @ W_pe2)
    b_row = w_pe @ W_pe2
    wedge_pad = jnp.zeros((8, 128), jnp.float32).at[:4].set(W_edge)
    mlm_wT_pad = jnp.zeros((128, 128), jnp.float32).at[:, :NUM_ATOM_TYPE].set(
        mlm_weight.T)
    mlm_bias_pad = jnp.zeros((1, 128), jnp.float32).at[0, :NUM_ATOM_TYPE].set(
        mlm_bias)

    atoms = xm0[src]
    du = u[src] - u[dst]
    pe2 = jnp.sum(du * du, axis=-1)
    dun = u_masked[src] - u_masked[dst]
    pn2 = jnp.sum(dun * dun, axis=-1)

    sdst = lut[dst]
    mm = sdst >= 0
    idx_safe = jnp.where(mm, sdst, _NM)
    cg0 = jnp.zeros((_NMP, 128), jnp.float32).at[idx_safe, atoms].add(
        jnp.where(mm, 1.0, 0.0))
    eam0 = jnp.zeros((_NMP, 8), jnp.float32).at[idx_safe, :4].add(
        jnp.where(mm, 1.0, 0.0)[:, None] * edge_attr)
    pnm0 = jnp.zeros((_NMP, 1), jnp.float32).at[idx_safe, 0].add(
        jnp.where(mm, jnp.sqrt(pn2), 0.0))
    cg = jnp.stack([cg0, jnp.zeros_like(cg0)])
    eam = jnp.stack([eam0, jnp.zeros_like(eam0)])
    pnm = jnp.stack([pnm0, jnp.zeros_like(pnm0)])

    snm = jnp.zeros((_NMP, 1), jnp.float32).at[:num_mask, 0].set(
        snorm_n[mask_nodes, 0])
    tgt = jnp.zeros((_NMP,), jnp.int32).at[:num_mask].set(x[mask_nodes, 0])

    v = _edge_chain(atoms, edge_attr, pe2, pn2, hp_pad, A, b_row, dh_dense_w,
                    dh_dense_b, dh_ln_g, dh_ln_b, dh_out_w, dh_out_b)[:, 0]
    atom_loss = _node_head(cg, eam, pnm, snm, wemb_pad, wedge_pad, w_pe,
                           W_gnn, mlm_dense_w, mlm_dense_b, mlm_ln_g,
                           mlm_ln_b, mlm_wT_pad, mlm_bias_pad, tgt)[0, 0]

    rows_i = jnp.concatenate([lut[src], lut[dst]])
    cols_i = jnp.concatenate([dst, src])
    vv = jnp.concatenate([v, v])
    msk = rows_i >= 0
    cells = jnp.where(msk, rows_i, 0) * N + cols_i
    sgrid = jnp.zeros((_NM * N,), jnp.float32).at[cells].add(
        jnp.where(msk, vv, 0.0))
    cgrid = jnp.zeros((_NM * N,), jnp.float32).at[cells].add(
        jnp.where(msk, 1.0, 0.0))
    sb = sgrid[cells]
    cb = jnp.maximum(cgrid[cells], 1.0)
    num = jnp.sum(jnp.where(msk, _huber(sb / cb) / cb, 0.0))
    den = jnp.sum(jnp.where(msk, 1.0 / cb, 0.0))
    pe_loss = num / den

    return atom_loss + pe_loss
